# baseline (speedup 1.0000x reference)
import functools

import jax
import jax.numpy as jnp
from jax import lax
from jax.experimental import pallas as pl
from jax.experimental.pallas import tpu as pltpu
from jax.experimental.pallas import tpu_sc as plsc

BATCH = 16384
NFIELD = 26
EMBED = 16
ROWS = BATCH * NFIELD
NW = 32
RPW = ROWS // NW
CH = 1664
NCH = RPW // CH

_mesh = plsc.VectorSubcoreMesh(core_axis_name="c", subcore_axis_name="s")


@functools.partial(
    pl.kernel,
    mesh=_mesh,
    out_type=jax.ShapeDtypeStruct((ROWS, EMBED), jnp.float32),
    scratch_types=[
        pltpu.VMEM((RPW,), jnp.int32),
        pltpu.VMEM((CH, EMBED), jnp.float32),
        pltpu.VMEM((CH, EMBED), jnp.float32),
        pltpu.VMEM((CH, EMBED), jnp.float32),
        pltpu.VMEM((CH, EMBED), jnp.float32),
        pltpu.SemaphoreType.DMA,
        pltpu.SemaphoreType.DMA,
    ],
    compiler_params=pltpu.CompilerParams(use_tc_tiling_on_sc=False),
)
def _sc_gather(idx_hbm, table_hbm, out_hbm, idx_v, b0, b1, b2, b3, sem_g, sem_w):
    wid = lax.axis_index("s") * 2 + lax.axis_index("c")
    base = wid * RPW
    pltpu.sync_copy(idx_hbm.at[pl.ds(base, RPW)], idx_v)

    bufs = (b0, b1, b2, b3)
    nbuf = len(bufs)

    def fire_gather(j):
        return pltpu.async_copy(
            table_hbm.at[idx_v.at[pl.ds(j * CH, CH)]], bufs[j % nbuf], sem_g)

    def fire_write(j):
        return pltpu.async_copy(
            bufs[j % nbuf], out_hbm.at[pl.ds(base + j * CH, CH)], sem_w)

    g = {j: fire_gather(j) for j in range(min(nbuf, NCH))}
    w = {}
    for j in range(NCH):
        g[j].wait()
        w[j] = fire_write(j)
        nj = j + nbuf
        if nj < NCH:
            w[j].wait()
            g[nj] = fire_gather(nj)
    for j in range(max(0, NCH - nbuf), NCH):
        w[j].wait()


NUM_EMB = 2600001
TW = 8192


def _tr_body(t_ref, o_ref):
    o_ref[...] = t_ref[...].T


def _idx_body(x_ref, o_ref):
    x = x_ref[...]
    f = lax.broadcasted_iota(jnp.int32, x.shape, 1) * 100000
    o_ref[...] = jnp.where(x == 0, 0, x + f)


def kernel(x, table):
    idx = pl.pallas_call(
        _idx_body,
        out_shape=jax.ShapeDtypeStruct((BATCH, NFIELD), jnp.int32),
    )(x)
    table_rm = pl.pallas_call(
        _tr_body,
        grid=(pl.cdiv(NUM_EMB, TW),),
        in_specs=[pl.BlockSpec((EMBED, TW), lambda i: (0, i))],
        out_specs=pl.BlockSpec((TW, EMBED), lambda i: (i, 0)),
        out_shape=jax.ShapeDtypeStruct((NUM_EMB, EMBED), jnp.float32),
    )(table.T)
    out = _sc_gather(idx.reshape(ROWS), table_rm)
    return out.reshape(BATCH, NFIELD, EMBED)


# speedup vs baseline: 2.4176x; 2.4176x over previous
import functools

import jax
import jax.numpy as jnp
from jax import lax
from jax.experimental import pallas as pl
from jax.experimental.pallas import tpu as pltpu
from jax.experimental.pallas import tpu_sc as plsc

BATCH = 16384
NFIELD = 26
EMBED = 16
ROWS = BATCH * NFIELD
NW = 32
RPW = ROWS // NW
CH = 1664
NCH = RPW // CH

_mesh = plsc.VectorSubcoreMesh(core_axis_name="c", subcore_axis_name="s")


@functools.partial(
    pl.kernel,
    mesh=_mesh,
    out_type=jax.ShapeDtypeStruct((ROWS, EMBED), jnp.float32),
    scratch_types=[
        pltpu.VMEM((RPW,), jnp.int32),
        pltpu.VMEM((CH, EMBED), jnp.float32),
        pltpu.VMEM((CH, EMBED), jnp.float32),
        pltpu.VMEM((CH, EMBED), jnp.float32),
        pltpu.VMEM((CH, EMBED), jnp.float32),
        pltpu.SemaphoreType.DMA,
        pltpu.SemaphoreType.DMA,
    ],
    compiler_params=pltpu.CompilerParams(use_tc_tiling_on_sc=False),
)
def _sc_gather(idx_hbm, table_hbm, out_hbm, idx_v, b0, b1, b2, b3, sem_g, sem_w):
    wid = lax.axis_index("s") * 2 + lax.axis_index("c")
    base = wid * RPW
    pltpu.sync_copy(idx_hbm.at[pl.ds(base, RPW)], idx_v)

    bufs = (b0, b1, b2, b3)
    nbuf = len(bufs)

    def fire_gather(j):
        return pltpu.async_copy(
            table_hbm.at[idx_v.at[pl.ds(j * CH, CH)]], bufs[j % nbuf], sem_g)

    def fire_write(j):
        return pltpu.async_copy(
            bufs[j % nbuf], out_hbm.at[pl.ds(base + j * CH, CH)], sem_w)

    g = {j: fire_gather(j) for j in range(min(nbuf, NCH))}
    w = {}
    for j in range(NCH):
        g[j].wait()
        w[j] = fire_write(j)
        nj = j + nbuf
        if nj < NCH:
            w[j].wait()
            g[nj] = fire_gather(nj)
    for j in range(max(0, NCH - nbuf), NCH):
        w[j].wait()


NUM_EMB = 2600001
TB = 8192
NBLK = (NUM_EMB + TB - 1) // TB
NPAD = NBLK * TB


def _tr_body(t_ref, o_ref):
    xw = jnp.concatenate(
        [t_ref[:, k * 1024:(k + 1) * 1024] for k in range(8)], axis=0)
    o_ref[...] = xw.T


def _idx_body(x_ref, o_ref):
    x = x_ref[...]
    f = lax.broadcasted_iota(jnp.int32, x.shape, 1) * 100000
    r = jnp.where(x == 0, 0, x + f)
    band = r & (TB - 1)
    o_ref[...] = (r - band) + ((band & 1023) << 3) + (band >> 10)


def kernel(x, table):
    idx = pl.pallas_call(
        _idx_body,
        out_shape=jax.ShapeDtypeStruct((BATCH, NFIELD), jnp.int32),
    )(x)
    table8 = pl.pallas_call(
        _tr_body,
        grid=(NBLK,),
        in_specs=[pl.BlockSpec((EMBED, TB), lambda i: (0, i))],
        out_specs=pl.BlockSpec((TB // 8, 128), lambda i: (i, 0)),
        out_shape=jax.ShapeDtypeStruct((NPAD // 8, 128), jnp.float32),
    )(table.T)
    out = _sc_gather(idx.reshape(ROWS), table8.reshape(NPAD, EMBED))
    return out.reshape(BATCH, NFIELD, EMBED)


# speedup vs baseline: 2.4667x; 1.0203x over previous
import functools

import jax
import jax.numpy as jnp
from jax import lax
from jax.experimental import pallas as pl
from jax.experimental.pallas import tpu as pltpu
from jax.experimental.pallas import tpu_sc as plsc

BATCH = 16384
NFIELD = 26
EMBED = 16
ROWS = BATCH * NFIELD
NW = 32
RPW = ROWS // NW
CH = 1664
NCH = RPW // CH

_mesh = plsc.VectorSubcoreMesh(core_axis_name="c", subcore_axis_name="s")


@functools.partial(
    pl.kernel,
    mesh=_mesh,
    out_type=jax.ShapeDtypeStruct((ROWS, EMBED), jnp.float32),
    scratch_types=[
        pltpu.VMEM((RPW,), jnp.int32),
        pltpu.VMEM((CH, EMBED), jnp.float32),
        pltpu.VMEM((CH, EMBED), jnp.float32),
        pltpu.VMEM((CH, EMBED), jnp.float32),
        pltpu.VMEM((CH, EMBED), jnp.float32),
        pltpu.SemaphoreType.DMA,
        pltpu.SemaphoreType.DMA,
    ],
    compiler_params=pltpu.CompilerParams(use_tc_tiling_on_sc=False),
)
def _sc_gather(idx_hbm, table_hbm, out_hbm, idx_v, b0, b1, b2, b3, sem_g, sem_w):
    wid = lax.axis_index("s") * 2 + lax.axis_index("c")
    base = wid * RPW
    pltpu.sync_copy(idx_hbm.at[pl.ds(base, RPW)], idx_v)

    bufs = (b0, b1, b2, b3)
    nbuf = len(bufs)

    def fire_gather(j):
        return pltpu.async_copy(
            table_hbm.at[idx_v.at[pl.ds(j * CH, CH)]], bufs[j % nbuf], sem_g)

    def fire_write(j):
        return pltpu.async_copy(
            bufs[j % nbuf], out_hbm.at[pl.ds(base + j * CH, CH)], sem_w)

    g = {j: fire_gather(j) for j in range(min(nbuf, NCH))}
    w = {}
    for j in range(NCH):
        g[j].wait()
        w[j] = fire_write(j)
        nj = j + nbuf
        if nj < NCH:
            w[j].wait()
            g[nj] = fire_gather(nj)
    for j in range(max(0, NCH - nbuf), NCH):
        w[j].wait()


NUM_EMB = 2600001
TB = 32768
KS = TB // 1024
NBLK = (NUM_EMB + TB - 1) // TB
NPAD = NBLK * TB


def _tr_body(t_ref, o_ref):
    xw = jnp.concatenate(
        [t_ref[:, k * 1024:(k + 1) * 1024] for k in range(KS)], axis=0)
    o_ref[...] = xw.T


def _idx_body(x_ref, o_ref):
    x = x_ref[...]
    f = lax.broadcasted_iota(jnp.int32, x.shape, 1) * 100000
    r = jnp.where(x == 0, 0, x + f)
    band = r & (TB - 1)
    o_ref[...] = (r - band) + (band & 1023) * KS + (band >> 10)


def kernel(x, table):
    idx = pl.pallas_call(
        _idx_body,
        out_shape=jax.ShapeDtypeStruct((BATCH, NFIELD), jnp.int32),
    )(x)
    table8 = pl.pallas_call(
        _tr_body,
        grid=(NBLK,),
        in_specs=[pl.BlockSpec((EMBED, TB), lambda i: (0, i))],
        out_specs=pl.BlockSpec((1024, 16 * KS), lambda i: (i, 0)),
        out_shape=jax.ShapeDtypeStruct((NBLK * 1024, 16 * KS), jnp.float32),
    )(table.T)
    out = _sc_gather(idx.reshape(ROWS), table8.reshape(NPAD, EMBED))
    return out.reshape(BATCH, NFIELD, EMBED)
